# baseline (speedup 1.0000x reference)
import functools

import jax
import jax.numpy as jnp
import numpy as np
from jax import lax
from jax.experimental import pallas as pl
from jax.experimental.pallas import tpu as pltpu
from jax.experimental.pallas import tpu_sc as plsc

Q, D, K = 676, 384, 65536
QP = 680
BK = 2048
NSTEP = K // BK
KF = K // 8
BKF = BK // 8
FMAP = 26
IMG = 224
KSIZE = 33
SIGMA = 4.0
BIGI = 2 ** 30

NC, NS, L = 2, 16, 16
NW = NC * NS
PERW = K // NW
NVR = PERW // L
NCAND = NW * 5 * L


def _ka_body(patch_ref, lib_ref,
             mind2_ref, bidx_ref, mtest_ref, sstar_ref,
             pn_ref, mx_ref, am_ref, riota_ref):
    k = pl.program_id(0)

    @pl.when(k == 0)
    def _init():
        p = patch_ref[...]
        nrm = jnp.sqrt(jnp.sum(p * p, axis=1, keepdims=True))
        pn_ref[...] = p / jnp.clip(nrm, 1e-12)
        mx_ref[...] = jnp.full((8, QP), -jnp.inf, jnp.float32)
        am_ref[...] = jnp.zeros((8, QP), jnp.int32)
        riota_ref[...] = lax.broadcasted_iota(jnp.int32, (BK, QP), 0)

    pn = pn_ref[...]
    lib = lib_ref[...]
    dt = lax.dot_general(lib, pn, (((1,), (1,)), ((), ())),
                         preferred_element_type=jnp.float32)
    tmax = jnp.max(dt, axis=0)
    targ = jnp.min(jnp.where(dt == tmax[None, :], riota_ref[...], BIGI),
                   axis=0) + k * BK
    cur = mx_ref[...]
    upd = tmax[None, :] > cur
    mx_ref[...] = jnp.where(upd, jnp.broadcast_to(tmax[None, :], (8, QP)), cur)
    am_ref[...] = jnp.where(upd, jnp.broadcast_to(targ[None, :], (8, QP)),
                            am_ref[...])

    @pl.when(k == NSTEP - 1)
    def _finish():
        a2 = jnp.sum(pn * pn, axis=1)
        md = a2 + 1.0 - 2.0 * mx_ref[0, :]
        mind2_ref[...] = jnp.broadcast_to(md[None, :], (8, QP))
        col = lax.broadcasted_iota(jnp.int32, (8, QP), 1)
        valid = col < Q
        minv = jnp.sqrt(jnp.clip(mind2_ref[...], 1e-12))
        mm = jnp.where(valid, minv, -jnp.inf)
        s_star = jnp.max(mm)
        s_idx = jnp.min(jnp.where(mm == s_star, col, BIGI))
        b_row = jnp.sum(jnp.where(col == s_idx, am_ref[...], 0), axis=1)
        bidx_ref[...] = jnp.broadcast_to(b_row[:, None], (8, 128))
        roh = lax.broadcasted_iota(jnp.int32, (QP, D), 0) == s_idx
        mt = jnp.sum(jnp.where(roh, pn, 0.0), axis=0)
        mtest_ref[...] = jnp.broadcast_to(mt[None, :], (8, D))
        sstar_ref[...] = jnp.full((8, 128), s_star, jnp.float32)


def _run_ka(patch_p, patch_lib, interpret=False):
    return pl.pallas_call(
        _ka_body,
        grid=(NSTEP,),
        in_specs=[
            pl.BlockSpec((QP, D), lambda k: (0, 0)),
            pl.BlockSpec((BK, D), lambda k: (k, 0)),
        ],
        out_specs=[
            pl.BlockSpec((8, QP), lambda k: (0, 0)),
            pl.BlockSpec((8, 128), lambda k: (0, 0)),
            pl.BlockSpec((8, D), lambda k: (0, 0)),
            pl.BlockSpec((8, 128), lambda k: (0, 0)),
        ],
        out_shape=[
            jax.ShapeDtypeStruct((8, QP), jnp.float32),
            jax.ShapeDtypeStruct((8, 128), jnp.int32),
            jax.ShapeDtypeStruct((8, D), jnp.float32),
            jax.ShapeDtypeStruct((8, 128), jnp.float32),
        ],
        scratch_shapes=[
            pltpu.VMEM((QP, D), jnp.float32),
            pltpu.VMEM((8, QP), jnp.float32),
            pltpu.VMEM((8, QP), jnp.int32),
            pltpu.VMEM((BK, QP), jnp.int32),
        ],
        interpret=interpret,
    )(patch_p, patch_lib)


def _kb_body(bidx_ref, lib_ref, bblk_ref, mtest_ref, dmf_ref, dqf_ref):
    k = pl.program_id(0)
    b = bidx_ref[0]
    r = b - (b // 8) * 8
    i0 = lax.broadcasted_iota(jnp.int32, (8, D), 0)
    m_star = jnp.sum(jnp.where(i0 == r, bblk_ref[...], 0.0), axis=0)
    w = jnp.where(i0 == 0, jnp.broadcast_to(m_star[None, :], (8, D)),
                  jnp.where(i0 == 1, mtest_ref[...], 0.0))
    a2 = jnp.sum(w * w, axis=1)
    dots = lax.dot_general(w, lib_ref[...], (((1,), (1,)), ((), ())),
                           preferred_element_type=jnp.float32)
    dd = a2[:, None] + 1.0 - 2.0 * dots
    off = pl.multiple_of(k * BKF, BKF)
    dmf_ref[:, pl.ds(off, BKF)] = dd[0, :].reshape(8, BKF)
    dqf_ref[:, pl.ds(off, BKF)] = dd[1, :].reshape(8, BKF)


def _run_kb(bidx1, patch_lib, mtest, interpret=False):
    grid_spec = pltpu.PrefetchScalarGridSpec(
        num_scalar_prefetch=1,
        grid=(NSTEP,),
        in_specs=[
            pl.BlockSpec((BK, D), lambda k, b: (k, 0)),
            pl.BlockSpec((8, D), lambda k, b: (b[0] // 8, 0)),
            pl.BlockSpec((8, D), lambda k, b: (0, 0)),
        ],
        out_specs=[
            pl.BlockSpec((8, KF), lambda k, b: (0, 0)),
            pl.BlockSpec((8, KF), lambda k, b: (0, 0)),
        ],
    )
    return pl.pallas_call(
        _kb_body,
        grid_spec=grid_spec,
        out_shape=[
            jax.ShapeDtypeStruct((8, KF), jnp.float32),
            jax.ShapeDtypeStruct((8, KF), jnp.float32),
        ],
        interpret=interpret,
    )(bidx1, patch_lib, patch_lib, mtest)


def _sc_topk_kernel(dm_hbm, dq_hbm, b_hbm,
                    candv_hbm, candj_hbm, candq_hbm, qb_hbm,
                    dm_v, dq_v, bv_v, cv_v, cj_v, cq_v, qb_v):
    c = lax.axis_index("c")
    s = lax.axis_index("s")
    wid = s * NC + c
    base = wid * PERW
    pltpu.sync_copy(dm_hbm.at[pl.ds(base, PERW)], dm_v)
    pltpu.sync_copy(dq_hbm.at[pl.ds(base, PERW)], dq_v)
    pltpu.sync_copy(b_hbm, bv_v)
    bv = bv_v[...]
    lane = lax.iota(jnp.int32, 16)
    inf = jnp.full((16,), jnp.inf, jnp.float32)
    zero = jnp.zeros((16,), jnp.float32)
    izero = jnp.zeros((16,), jnp.int32)
    init = (inf, inf, inf, inf, inf,
            izero, izero, izero, izero, izero,
            zero, zero, zero, zero, zero,
            zero)

    def body(i, carry):
        t = list(carry[0:5])
        tj = list(carry[5:10])
        tq = list(carry[10:15])
        qb = carry[15]
        v = dm_v[pl.ds(i * L, L)]
        q = dq_v[pl.ds(i * L, L)]
        f = base + i * L + lane
        r = lax.shift_right_logical(f, 13)
        cc = jnp.bitwise_and(f, KF - 1)
        j = (lax.shift_left(lax.shift_right_logical(cc, 8), 11)
             + lax.shift_left(r, 8) + jnp.bitwise_and(cc, BKF - 1))
        qb = qb + jnp.where(j == bv, q, 0.0)
        for lev in range(5):
            sw = v < t[lev]
            t[lev], v = jnp.where(sw, v, t[lev]), jnp.where(sw, t[lev], v)
            tj[lev], j = jnp.where(sw, j, tj[lev]), jnp.where(sw, tj[lev], j)
            tq[lev], q = jnp.where(sw, q, tq[lev]), jnp.where(sw, tq[lev], q)
        return tuple(t) + tuple(tj) + tuple(tq) + (qb,)

    out = lax.fori_loop(0, NVR, body, init)
    for lev in range(5):
        cv_v[pl.ds(lev * L, L)] = out[lev]
        cj_v[pl.ds(lev * L, L)] = out[5 + lev]
        cq_v[pl.ds(lev * L, L)] = out[10 + lev]
    qb_v[...] = out[15]
    pltpu.sync_copy(cv_v, candv_hbm.at[pl.ds(wid * 5 * L, 5 * L)])
    pltpu.sync_copy(cj_v, candj_hbm.at[pl.ds(wid * 5 * L, 5 * L)])
    pltpu.sync_copy(cq_v, candq_hbm.at[pl.ds(wid * 5 * L, 5 * L)])
    pltpu.sync_copy(qb_v, qb_hbm.at[pl.ds(wid * L, L)])


def _run_sc(dmflat, dqflat, bvec):
    mesh = plsc.VectorSubcoreMesh(core_axis_name="c", subcore_axis_name="s")
    kfn = functools.partial(
        pl.kernel, mesh=mesh,
        out_type=[
            jax.ShapeDtypeStruct((NCAND,), jnp.float32),
            jax.ShapeDtypeStruct((NCAND,), jnp.int32),
            jax.ShapeDtypeStruct((NCAND,), jnp.float32),
            jax.ShapeDtypeStruct((NW * L,), jnp.float32),
        ],
        scratch_types=[
            pltpu.VMEM((PERW,), jnp.float32),
            pltpu.VMEM((PERW,), jnp.float32),
            pltpu.VMEM((L,), jnp.int32),
            pltpu.VMEM((5 * L,), jnp.float32),
            pltpu.VMEM((5 * L,), jnp.int32),
            pltpu.VMEM((5 * L,), jnp.float32),
            pltpu.VMEM((L,), jnp.float32),
        ],
    )(_sc_topk_kernel)
    return kfn(dmflat, dqflat, bvec)


def _kc_body(cv_ref, cj_ref, cq_ref, qb_ref, v26_ref, sstar_ref,
             rmat_ref, bmat_ref, s_ref, smap_ref):
    cv = cv_ref[...]
    cj = cj_ref[...]
    cq = cq_ref[...]
    den = jnp.float32(0.0)
    for _ in range(5):
        m = jnp.min(cv)
        sel = jnp.min(jnp.where(cv == m, cj, BIGI))
        oh = cj == sel
        dq = jnp.sum(jnp.where(oh, cq, 0.0))
        den = den + jnp.exp(jnp.sqrt(jnp.clip(dq, 0.0)))
        cv = jnp.where(oh, jnp.inf, cv)
    dqb = jnp.sum(qb_ref[...])
    num = jnp.exp(jnp.sqrt(jnp.clip(dqb, 0.0)))
    s_star = sstar_ref[0, 0]
    s_ref[...] = jnp.full((8, 128), (1.0 - num / den) * s_star, jnp.float32)

    v = jnp.sqrt(jnp.clip(v26_ref[...], 1e-12))
    rm = rmat_ref[...]
    bm = bmat_ref[...]
    t1 = lax.dot_general(rm, v, (((1,), (0,)), ((), ())),
                         preferred_element_type=jnp.float32)
    t2 = lax.dot_general(t1, rm, (((1,), (1,)), ((), ())),
                         preferred_element_type=jnp.float32)
    t3 = lax.dot_general(bm, t2, (((1,), (0,)), ((), ())),
                         preferred_element_type=jnp.float32)
    smap_ref[...] = lax.dot_general(t3, bm, (((1,), (1,)), ((), ())),
                                    preferred_element_type=jnp.float32)


def _run_kc(cv, cj, cq, qb, v26, sstar, rmat, bmat, interpret=False):
    return pl.pallas_call(
        _kc_body,
        out_shape=[
            jax.ShapeDtypeStruct((8, 128), jnp.float32),
            jax.ShapeDtypeStruct((IMG, IMG), jnp.float32),
        ],
        interpret=interpret,
    )(cv, cj, cq, qb, v26, sstar, rmat, bmat)


def _blur_matrix():
    ax = np.arange(KSIZE, dtype=np.float32) - (KSIZE // 2)
    g = np.exp(-(ax ** 2) / (2.0 * SIGMA ** 2))
    g = g / np.sum(g)
    pad = KSIZE // 2
    eye = np.eye(IMG, dtype=np.float32)
    xp = np.pad(eye, ((pad, pad), (0, 0)), mode="reflect")
    b = np.zeros((IMG, IMG), dtype=np.float32)
    for t in range(KSIZE):
        b += g[t] * xp[t:t + IMG, :]
    return b


_BMAT = _blur_matrix()


def _resize_matrix():
    return jax.image.resize(jnp.eye(FMAP, dtype=jnp.float32), (IMG, FMAP),
                            method="bilinear")


def _kernel_impl(patch, patch_lib, interpret=False):
    patch_p = jnp.zeros((QP, D), jnp.float32).at[:Q].set(patch)
    mind2, bidx, mtest, sstar = _run_ka(patch_p, patch_lib, interpret)
    bidx1 = bidx[0, 0].reshape(1).astype(jnp.int32)
    v26 = mind2[0, :Q].reshape(FMAP, FMAP)
    dmf, dqf = _run_kb(bidx1, patch_lib, mtest, interpret)
    bvec = jnp.broadcast_to(bidx[0, 0], (16,)).astype(jnp.int32)
    cv, cj, cq, qb = _run_sc(dmf.reshape(K), dqf.reshape(K), bvec)
    s_out, smap = _run_kc(cv.reshape(20, 128), cj.reshape(20, 128),
                          cq.reshape(20, 128), qb.reshape(4, 128),
                          v26, sstar, _resize_matrix(), jnp.asarray(_BMAT),
                          interpret)
    return s_out[0, 0], smap.reshape(1, 1, IMG, IMG)


def kernel(patch, patch_lib):
    return _kernel_impl(patch, patch_lib, interpret=False)


# speedup vs baseline: 1.2589x; 1.2589x over previous
import jax
import jax.numpy as jnp
import numpy as np
from jax import lax
from jax.experimental import pallas as pl
from jax.experimental.pallas import tpu as pltpu

Q, D, K = 676, 384, 65536
QP = 680
BK = 4096
NSTEP = K // BK
FMAP = 26
IMG = 224
KSIZE = 33
SIGMA = 4.0
BIGI = 2 ** 30


def _ka_body(patch_ref, lib_ref,
             mind2_ref, bidx_ref, mtest_ref, sstar_ref,
             pn_ref, mx_ref, am_ref, riota_ref):
    k = pl.program_id(0)

    @pl.when(k == 0)
    def _init():
        p = patch_ref[...]
        nrm = jnp.sqrt(jnp.sum(p * p, axis=1, keepdims=True))
        pn_ref[...] = p / jnp.clip(nrm, 1e-12)
        mx_ref[...] = jnp.full((8, QP), -jnp.inf, jnp.float32)
        am_ref[...] = jnp.zeros((8, QP), jnp.int32)
        riota_ref[...] = lax.broadcasted_iota(jnp.int32, (BK, QP), 0)

    pn = pn_ref[...]
    lib = lib_ref[...]
    dt = lax.dot_general(lib, pn, (((1,), (1,)), ((), ())),
                         preferred_element_type=jnp.float32)
    tmax = jnp.max(dt, axis=0)
    targ = jnp.min(jnp.where(dt == tmax[None, :], riota_ref[...], BIGI),
                   axis=0) + k * BK
    cur = mx_ref[...]
    upd = tmax[None, :] > cur
    mx_ref[...] = jnp.where(upd, jnp.broadcast_to(tmax[None, :], (8, QP)), cur)
    am_ref[...] = jnp.where(upd, jnp.broadcast_to(targ[None, :], (8, QP)),
                            am_ref[...])

    @pl.when(k == NSTEP - 1)
    def _finish():
        a2 = jnp.sum(pn * pn, axis=1)
        md = a2 + 1.0 - 2.0 * mx_ref[0, :]
        mind2_ref[...] = jnp.broadcast_to(md[None, :], (8, QP))
        col = lax.broadcasted_iota(jnp.int32, (8, QP), 1)
        valid = col < Q
        minv = jnp.sqrt(jnp.clip(mind2_ref[...], 1e-12))
        mm = jnp.where(valid, minv, -jnp.inf)
        s_star = jnp.max(mm)
        s_idx = jnp.min(jnp.where(mm == s_star, col, BIGI))
        b_row = jnp.sum(jnp.where(col == s_idx, am_ref[...], 0), axis=1)
        bidx_ref[...] = jnp.broadcast_to(b_row[:, None], (8, 128))
        roh = lax.broadcasted_iota(jnp.int32, (QP, D), 0) == s_idx
        mt = jnp.sum(jnp.where(roh, pn, 0.0), axis=0)
        mtest_ref[...] = jnp.broadcast_to(mt[None, :], (8, D))
        sstar_ref[...] = jnp.full((8, 128), s_star, jnp.float32)


def _run_ka(patch_p, patch_lib, interpret=False):
    return pl.pallas_call(
        _ka_body,
        grid=(NSTEP,),
        in_specs=[
            pl.BlockSpec((QP, D), lambda k: (0, 0)),
            pl.BlockSpec((BK, D), lambda k: (k, 0)),
        ],
        out_specs=[
            pl.BlockSpec((8, QP), lambda k: (0, 0)),
            pl.BlockSpec((8, 128), lambda k: (0, 0)),
            pl.BlockSpec((8, D), lambda k: (0, 0)),
            pl.BlockSpec((8, 128), lambda k: (0, 0)),
        ],
        out_shape=[
            jax.ShapeDtypeStruct((8, QP), jnp.float32),
            jax.ShapeDtypeStruct((8, 128), jnp.int32),
            jax.ShapeDtypeStruct((8, D), jnp.float32),
            jax.ShapeDtypeStruct((8, 128), jnp.float32),
        ],
        scratch_shapes=[
            pltpu.VMEM((QP, D), jnp.float32),
            pltpu.VMEM((8, QP), jnp.float32),
            pltpu.VMEM((8, QP), jnp.int32),
            pltpu.VMEM((BK, QP), jnp.int32),
        ],
        interpret=interpret,
    )(patch_p, patch_lib)


KF = K // 8
BKF = BK // 8


def _kb_body(bidx_ref, lib_ref, bblk_ref, mtest_ref, v26_ref, sstar_ref,
             rmat_ref, bmat_ref, s_ref, smap_ref, dmf_ref, dqf_ref, jio_ref):
    k = pl.program_id(0)
    b = bidx_ref[0]
    r = b - (b // 8) * 8
    i0 = lax.broadcasted_iota(jnp.int32, (8, D), 0)
    m_star = jnp.sum(jnp.where(i0 == r, bblk_ref[...], 0.0), axis=0)
    w = jnp.where(i0 == 0, jnp.broadcast_to(m_star[None, :], (8, D)),
                  jnp.where(i0 == 1, mtest_ref[...], 0.0))
    a2 = jnp.sum(w * w, axis=1)
    dots = lax.dot_general(w, lib_ref[...], (((1,), (1,)), ((), ())),
                           preferred_element_type=jnp.float32)
    dd = a2[:, None] + 1.0 - 2.0 * dots
    off = pl.multiple_of(k * BKF, BKF)
    dmf_ref[:, pl.ds(off, BKF)] = dd[0, :].reshape(8, BKF)
    dqf_ref[:, pl.ds(off, BKF)] = dd[1, :].reshape(8, BKF)

    @pl.when(k == 0)
    def _initjio():
        row = lax.broadcasted_iota(jnp.int32, (8, KF), 0)
        col = lax.broadcasted_iota(jnp.int32, (8, KF), 1)
        jio_ref[...] = (col // BKF) * BK + row * BKF + col % BKF

    @pl.when(k == NSTEP - 1)
    def _finish():
        dm = dmf_ref[...]
        dqf = dqf_ref[...]
        jio = jio_ref[...]
        den = jnp.float32(0.0)
        for _ in range(5):
            m = jnp.min(dm)
            sel = jnp.min(jnp.where(dm == m, jio, BIGI))
            oh = jio == sel
            dq = jnp.sum(jnp.where(oh, dqf, 0.0))
            den = den + jnp.exp(jnp.sqrt(jnp.clip(dq, 0.0)))
            dm = jnp.where(oh, jnp.inf, dm)

        dqb = jnp.sum(jnp.where(jio == b, dqf, 0.0))
        num = jnp.exp(jnp.sqrt(jnp.clip(dqb, 0.0)))
        s_star = sstar_ref[0, 0]
        s_ref[...] = jnp.full((8, 128), (1.0 - num / den) * s_star,
                              jnp.float32)

        v = jnp.sqrt(jnp.clip(v26_ref[...], 1e-12))
        rm = rmat_ref[...]
        bm = bmat_ref[...]
        t1 = lax.dot_general(rm, v, (((1,), (0,)), ((), ())),
                             preferred_element_type=jnp.float32)
        t2 = lax.dot_general(t1, rm, (((1,), (1,)), ((), ())),
                             preferred_element_type=jnp.float32)
        t3 = lax.dot_general(bm, t2, (((1,), (0,)), ((), ())),
                             preferred_element_type=jnp.float32)
        smap_ref[...] = lax.dot_general(t3, bm, (((1,), (1,)), ((), ())),
                                        preferred_element_type=jnp.float32)


def _run_kb(bidx1, patch_lib, mtest, v26, sstar, rmat, bmat, interpret=False):
    grid_spec = pltpu.PrefetchScalarGridSpec(
        num_scalar_prefetch=1,
        grid=(NSTEP,),
        in_specs=[
            pl.BlockSpec((BK, D), lambda k, b: (k, 0)),
            pl.BlockSpec((8, D), lambda k, b: (b[0] // 8, 0)),
            pl.BlockSpec((8, D), lambda k, b: (0, 0)),
            pl.BlockSpec((FMAP, FMAP), lambda k, b: (0, 0)),
            pl.BlockSpec((8, 128), lambda k, b: (0, 0)),
            pl.BlockSpec((IMG, FMAP), lambda k, b: (0, 0)),
            pl.BlockSpec((IMG, IMG), lambda k, b: (0, 0)),
        ],
        out_specs=[
            pl.BlockSpec((8, 128), lambda k, b: (0, 0)),
            pl.BlockSpec((IMG, IMG), lambda k, b: (0, 0)),
        ],
        scratch_shapes=[
            pltpu.VMEM((8, KF), jnp.float32),
            pltpu.VMEM((8, KF), jnp.float32),
            pltpu.VMEM((8, KF), jnp.int32),
        ],
    )
    return pl.pallas_call(
        _kb_body,
        grid_spec=grid_spec,
        out_shape=[
            jax.ShapeDtypeStruct((8, 128), jnp.float32),
            jax.ShapeDtypeStruct((IMG, IMG), jnp.float32),
        ],
        interpret=interpret,
    )(bidx1, patch_lib, patch_lib, mtest, v26, sstar, rmat, bmat)


def _blur_matrix():
    ax = np.arange(KSIZE, dtype=np.float32) - (KSIZE // 2)
    g = np.exp(-(ax ** 2) / (2.0 * SIGMA ** 2))
    g = g / np.sum(g)
    pad = KSIZE // 2
    eye = np.eye(IMG, dtype=np.float32)
    xp = np.pad(eye, ((pad, pad), (0, 0)), mode="reflect")
    b = np.zeros((IMG, IMG), dtype=np.float32)
    for t in range(KSIZE):
        b += g[t] * xp[t:t + IMG, :]
    return b


_BMAT = _blur_matrix()


def _resize_matrix():
    return jax.image.resize(jnp.eye(FMAP, dtype=jnp.float32), (IMG, FMAP),
                            method="bilinear")


def _kernel_impl(patch, patch_lib, interpret=False):
    patch_p = jnp.zeros((QP, D), jnp.float32).at[:Q].set(patch)
    mind2, bidx, mtest, sstar = _run_ka(patch_p, patch_lib, interpret)
    bidx1 = bidx[0, 0].reshape(1).astype(jnp.int32)
    v26 = mind2[0, :Q].reshape(FMAP, FMAP)
    s_out, smap = _run_kb(bidx1, patch_lib, mtest, v26, sstar,
                          _resize_matrix(), jnp.asarray(_BMAT), interpret)
    return s_out[0, 0], smap.reshape(1, 1, IMG, IMG)


def kernel(patch, patch_lib):
    return _kernel_impl(patch, patch_lib, interpret=False)
